# baseline (speedup 1.0000x reference)
import jax
import jax.numpy as jnp
from jax import lax
from jax.experimental import pallas as pl
from jax.experimental.pallas import tpu as pltpu
from jax.experimental.pallas import tpu_sc as plsc

N_ROWS = 64
C = 256
_HW = (56 * 56, 28 * 28, 14 * 14, 7 * 7, 4 * 4)


def _sc_body(match_hbm, rows_hbm, f0, f1, f2, f3, f4, a0, a1, a2, a3, a4,
             match_v, rows_v, fill_v0, fill_v1, fill_v2, fill_v3, fill_v4,
             sem):
    acts = (a0, a1, a2, a3, a4)
    fills = (fill_v0, fill_v1, fill_v2, fill_v3, fill_v4)
    fills_hbm = (f0, f1, f2, f3, f4)
    wid = lax.axis_index("s") * 2 + lax.axis_index("c")

    pltpu.sync_copy(match_hbm, match_v)
    pltpu.sync_copy(rows_hbm, rows_v)
    for li in range(5):
        pltpu.sync_copy(fills_hbm[li], fills[li])

    for jj in range(2):
        j = wid * 2 + jj
        lid_vec = match_v[j]
        row_vec = rows_v[j]
        r = row_vec[0].astype(jnp.int32)
        m = lid_vec[0]
        for li in range(5):
            @pl.when(m == float(li))
            def _(li=li, r=r):
                pltpu.async_copy(fills[li], acts[li].at[r], sem).wait()


def kernel(act_0, act_1, act_2, act_3, act_pool, indices, x):
    del x
    acts = (act_0, act_1, act_2, act_3, act_pool)
    layer_ids = (indices // C).astype(jnp.int32)
    ch = (indices % C).astype(jnp.int32)
    flat_rows = (jnp.arange(N_ROWS, dtype=jnp.int32) * C + ch)
    match = jnp.broadcast_to(layer_ids[:, None], (N_ROWS, 16)).astype(jnp.float32)
    rows = jnp.broadcast_to(flat_rows[:, None], (N_ROWS, 16)).astype(jnp.float32)
    fill = [jnp.full((hw,), -1000.0, jnp.float32) for hw in _HW]

    flat = [a.reshape(N_ROWS * C, hw) for a, hw in zip(acts, _HW)]

    mesh = plsc.VectorSubcoreMesh(core_axis_name="c", subcore_axis_name="s")
    sc_scatter = pl.kernel(
        _sc_body,
        out_type=(),
        mesh=mesh,
        scratch_types=[
            pltpu.VMEM((N_ROWS, 16), jnp.float32),
            pltpu.VMEM((N_ROWS, 16), jnp.float32),
        ] + [pltpu.VMEM((hw,), jnp.float32) for hw in _HW]
        + [pltpu.SemaphoreType.DMA],
    )

    refs = [jax.new_ref(f) for f in flat]
    sc_scatter(match, rows, *fill, *refs)
    outs = [r[...] for r in refs]
    return tuple(o.reshape(a.shape) for o, a in zip(outs, acts))


# speedup vs baseline: 1.1336x; 1.1336x over previous
import jax
import jax.numpy as jnp
from jax import lax
from jax.experimental import pallas as pl
from jax.experimental.pallas import tpu as pltpu
from jax.experimental.pallas import tpu_sc as plsc

N_ROWS = 64
C = 256
_HW = (56 * 56, 28 * 28, 14 * 14, 7 * 7, 4 * 4)
_BLOCK = ((4, 256), (16, 256), (32, 256), (64, 256), (64, 256))


def _copy_body(ain, aout):
    aout[...] = ain[...]


def _stream_copy(li, flat):
    rpb, cpb = _BLOCK[li]
    spec = pl.BlockSpec((rpb, cpb, _HW[li]), lambda b0, b1: (b0, b1, 0))
    return pl.pallas_call(
        _copy_body,
        grid=(N_ROWS // rpb, C // cpb),
        in_specs=[spec],
        out_specs=spec,
        out_shape=jax.ShapeDtypeStruct(flat.shape, flat.dtype),
    )(flat)


def _sc_body(match_hbm, rows_hbm, f0, f1, f2, f3, f4, a0, a1, a2, a3, a4,
             match_v, rows_v, fill_v0, fill_v1, fill_v2, fill_v3, fill_v4,
             sem):
    acts = (a0, a1, a2, a3, a4)
    fills = (fill_v0, fill_v1, fill_v2, fill_v3, fill_v4)
    fills_hbm = (f0, f1, f2, f3, f4)
    wid = lax.axis_index("s") * 2 + lax.axis_index("c")

    pltpu.sync_copy(match_hbm, match_v)
    pltpu.sync_copy(rows_hbm, rows_v)
    for li in range(5):
        pltpu.sync_copy(fills_hbm[li], fills[li])

    for jj in range(2):
        j = wid * 2 + jj
        lid_vec = match_v[j]
        row_vec = rows_v[j]
        r = row_vec[0].astype(jnp.int32)
        m = lid_vec[0]
        for li in range(5):
            @pl.when(m == float(li))
            def _(li=li, r=r):
                pltpu.async_copy(fills[li], acts[li].at[r], sem).wait()


def kernel(act_0, act_1, act_2, act_3, act_pool, indices, x):
    del x
    acts = (act_0, act_1, act_2, act_3, act_pool)
    layer_ids = (indices // C).astype(jnp.int32)
    ch = (indices % C).astype(jnp.int32)
    flat_rows = (jnp.arange(N_ROWS, dtype=jnp.int32) * C + ch)
    match = jnp.broadcast_to(layer_ids[:, None], (N_ROWS, 16))
    match = match.astype(jnp.float32)
    rows = jnp.broadcast_to(flat_rows[:, None], (N_ROWS, 16))
    rows = rows.astype(jnp.float32)
    fill = [jnp.full((hw,), -1000.0, jnp.float32) for hw in _HW]

    copies = [
        _stream_copy(li, a.reshape(N_ROWS, C, hw)).reshape(N_ROWS * C, hw)
        for li, (a, hw) in enumerate(zip(acts, _HW))
    ]

    mesh = plsc.VectorSubcoreMesh(core_axis_name="c", subcore_axis_name="s")
    sc_scatter = pl.kernel(
        _sc_body,
        out_type=(),
        mesh=mesh,
        scratch_types=[
            pltpu.VMEM((N_ROWS, 16), jnp.float32),
            pltpu.VMEM((N_ROWS, 16), jnp.float32),
        ] + [pltpu.VMEM((hw,), jnp.float32) for hw in _HW]
        + [pltpu.SemaphoreType.DMA],
    )

    refs = [jax.new_ref(f) for f in copies]
    sc_scatter(match, rows, *fill, *refs)
    outs = [r[...] for r in refs]
    return tuple(o.reshape(a.shape) for o, a in zip(outs, acts))
